# baseline (speedup 1.0000x reference)
import jax
import jax.numpy as jnp
from jax.experimental import pallas as pl
from jax.experimental.pallas import tpu as pltpu


def _dma_copy_body(e_hbm, o_hbm, sem):
    copy = pltpu.make_async_copy(e_hbm, o_hbm, sem)
    copy.start()
    copy.wait()


def kernel(emb_h, h, e, conv_w, conv_b, w2, b2, edge_index):
    E, H = e.shape
    out = pl.pallas_call(
        _dma_copy_body,
        in_specs=[pl.BlockSpec(memory_space=pl.ANY)],
        out_specs=pl.BlockSpec(memory_space=pl.ANY),
        out_shape=jax.ShapeDtypeStruct((E, H), e.dtype),
        scratch_shapes=[pltpu.SemaphoreType.DMA],
    )(e)
    return out


# speedup vs baseline: 1.0003x; 1.0003x over previous
import jax
import jax.numpy as jnp
from jax.experimental import pallas as pl
from jax.experimental.pallas import tpu as pltpu


_NCHUNK = 32


def _dma_copy_body(e_hbm, o_hbm, sem):
    n_rows = e_hbm.shape[0]
    rows_per = n_rows // _NCHUNK
    copies = []
    for i in range(_NCHUNK):
        sl = pl.ds(i * rows_per, rows_per)
        copies.append(pltpu.make_async_copy(e_hbm.at[sl], o_hbm.at[sl], sem))
    for c in copies:
        c.start()
    for c in copies:
        c.wait()


def kernel(emb_h, h, e, conv_w, conv_b, w2, b2, edge_index):
    E, H = e.shape
    out = pl.pallas_call(
        _dma_copy_body,
        in_specs=[pl.BlockSpec(memory_space=pl.ANY)],
        out_specs=pl.BlockSpec(memory_space=pl.ANY),
        out_shape=jax.ShapeDtypeStruct((E, H), e.dtype),
        scratch_shapes=[pltpu.SemaphoreType.DMA],
    )(e)
    return out


# speedup vs baseline: 48.2141x; 48.2009x over previous
import jax
import jax.numpy as jnp
from jax.experimental import pallas as pl
from jax.experimental.pallas import tpu as pltpu

_BLOCK_ROWS = 8000


def _copy_body(e_ref, o_ref):
    o_ref[...] = e_ref[...]


def kernel(emb_h, h, e, conv_w, conv_b, w2, b2, edge_index):
    E, H = e.shape
    block_rows = _BLOCK_ROWS if E % _BLOCK_ROWS == 0 else E
    grid = (E // block_rows,)
    out = pl.pallas_call(
        _copy_body,
        grid=grid,
        in_specs=[pl.BlockSpec((block_rows, H), lambda i: (i, 0))],
        out_specs=pl.BlockSpec((block_rows, H), lambda i: (i, 0)),
        out_shape=jax.ShapeDtypeStruct((E, H), e.dtype),
    )(e)
    return out


# speedup vs baseline: 48.9085x; 1.0144x over previous
import jax
import jax.numpy as jnp
from jax.experimental import pallas as pl
from jax.experimental.pallas import tpu as pltpu

_BLOCK_ROWS = 16000


def _copy_body(e_ref, o_ref):
    o_ref[...] = e_ref[...]


def kernel(emb_h, h, e, conv_w, conv_b, w2, b2, edge_index):
    E, H = e.shape
    block_rows = _BLOCK_ROWS if E % _BLOCK_ROWS == 0 else E
    grid = (E // block_rows,)
    out = pl.pallas_call(
        _copy_body,
        grid=grid,
        in_specs=[pl.BlockSpec((block_rows, H), lambda i: (i, 0))],
        out_specs=pl.BlockSpec((block_rows, H), lambda i: (i, 0)),
        out_shape=jax.ShapeDtypeStruct((E, H), e.dtype),
    )(e)
    return out


# speedup vs baseline: 49.0755x; 1.0034x over previous
import jax
import jax.numpy as jnp
from jax.experimental import pallas as pl
from jax.experimental.pallas import tpu as pltpu

_BLOCK_ROWS = 20000


def _copy_body(e_ref, o_ref):
    o_ref[...] = e_ref[...]


def kernel(emb_h, h, e, conv_w, conv_b, w2, b2, edge_index):
    E, H = e.shape
    block_rows = _BLOCK_ROWS if E % _BLOCK_ROWS == 0 else E
    grid = (E // block_rows,)
    out = pl.pallas_call(
        _copy_body,
        grid=grid,
        in_specs=[pl.BlockSpec((block_rows, H), lambda i: (i, 0))],
        out_specs=pl.BlockSpec((block_rows, H), lambda i: (i, 0)),
        out_shape=jax.ShapeDtypeStruct((E, H), e.dtype),
    )(e)
    return out
